# baseline (speedup 1.0000x reference)
import jax
import jax.numpy as jnp
from jax.experimental import pallas as pl
from jax.experimental.pallas import tpu as pltpu

_TT = 8


def _fused_kernel(eids_ref, regions_ref, spikes_ref, wfull_ref, bcol_ref,
                  slott_ref, regembt_ref, eidembt_ref, out_ref, base_ref):
    b_idx = pl.program_id(0)
    t_idx = pl.program_id(1)
    d, n = base_ref.shape

    @pl.when(t_idx == 0)
    def _build_base():
        regions = regions_ref[0, :, :]
        nregions = regembt_ref.shape[1]
        oht = (jax.lax.broadcasted_iota(jnp.int32, (nregions, n), 0)
               == regions).astype(jnp.float32)
        regt = jnp.dot(regembt_ref[...], oht,
                       preferred_element_type=jnp.float32)

        e = eids_ref[b_idx]
        neids = eidembt_ref.shape[1]
        ohe = (jax.lax.broadcasted_iota(jnp.int32, (neids, 8), 0) == e
               ).astype(jnp.float32)
        evt = jnp.dot(eidembt_ref[...], ohe,
                      preferred_element_type=jnp.float32)

        base_ref[...] = (slott_ref[...] + regt
                         + evt[:, 0:1] + bcol_ref[...])

    sp = spikes_ref[0]
    tt = sp.shape[0]
    out_ref[0] = (sp[:, None, :] * wfull_ref[...][None, :, :]
                  + base_ref[...][None, :, :])


@jax.jit
def kernel(spikes, neuron_regions, eids, w_spike, b_spike, neuron_slot,
           region_emb, eid_emb):
    B, T, N = spikes.shape
    D = neuron_slot.shape[1]
    regions3 = neuron_regions.astype(jnp.int32).reshape(B, 1, N)
    eids32 = eids.astype(jnp.int32)
    wfull = jnp.broadcast_to(w_spike, (D, N))
    bcol = b_spike.reshape(D, 1)
    slott = neuron_slot[:N].T
    regembt = region_emb.T
    eidembt = eid_emb.T

    outt = pl.pallas_call(
        _fused_kernel,
        grid=(B, T // _TT),
        in_specs=[
            pl.BlockSpec(memory_space=pltpu.SMEM),
            pl.BlockSpec((1, 1, N), lambda b, t: (b, 0, 0)),
            pl.BlockSpec((1, _TT, N), lambda b, t: (b, t, 0)),
            pl.BlockSpec((D, N), lambda b, t: (0, 0)),
            pl.BlockSpec((D, 1), lambda b, t: (0, 0)),
            pl.BlockSpec((D, N), lambda b, t: (0, 0)),
            pl.BlockSpec((D, region_emb.shape[0]), lambda b, t: (0, 0)),
            pl.BlockSpec((D, eid_emb.shape[0]), lambda b, t: (0, 0)),
        ],
        out_specs=pl.BlockSpec((1, _TT, D, N), lambda b, t: (b, t, 0, 0)),
        out_shape=jax.ShapeDtypeStruct((B, T, D, N), jnp.float32),
        scratch_shapes=[pltpu.VMEM((D, N), jnp.float32)],
    )(eids32, regions3, spikes, wfull, bcol, slott, regembt, eidembt)
    return outt.transpose(0, 1, 3, 2)


# speedup vs baseline: 1.3306x; 1.3306x over previous
import jax
import jax.numpy as jnp
from jax.experimental import pallas as pl
from jax.experimental.pallas import tpu as pltpu

_TT = 64


def _fused_kernel(eids_ref, regions_ref, spikes_ref, wfull_ref, bcol_ref,
                  slott_ref, regembt_ref, eidembt_ref, out_ref, base_ref):
    b_idx = pl.program_id(0)
    t_idx = pl.program_id(1)
    d, n = base_ref.shape

    @pl.when(t_idx == 0)
    def _build_base():
        regions = regions_ref[0, :, :]
        nregions = regembt_ref.shape[1]
        oht = (jax.lax.broadcasted_iota(jnp.int32, (nregions, n), 0)
               == regions).astype(jnp.float32)
        regt = jnp.dot(regembt_ref[...], oht,
                       preferred_element_type=jnp.float32)

        e = eids_ref[b_idx]
        neids = eidembt_ref.shape[1]
        ohe = (jax.lax.broadcasted_iota(jnp.int32, (neids, 8), 0) == e
               ).astype(jnp.float32)
        evt = jnp.dot(eidembt_ref[...], ohe,
                      preferred_element_type=jnp.float32)

        base_ref[...] = (slott_ref[...] + regt
                         + evt[:, 0:1] + bcol_ref[...])

    sp = spikes_ref[0]
    tt = sp.shape[0]
    out_ref[0] = (sp[:, None, :] * wfull_ref[...][None, :, :]
                  + base_ref[...][None, :, :])


@jax.jit
def kernel(spikes, neuron_regions, eids, w_spike, b_spike, neuron_slot,
           region_emb, eid_emb):
    B, T, N = spikes.shape
    D = neuron_slot.shape[1]
    regions3 = neuron_regions.astype(jnp.int32).reshape(B, 1, N)
    eids32 = eids.astype(jnp.int32)
    wfull = jnp.broadcast_to(w_spike, (D, N))
    bcol = b_spike.reshape(D, 1)
    slott = neuron_slot[:N].T
    regembt = region_emb.T
    eidembt = eid_emb.T

    outt = pl.pallas_call(
        _fused_kernel,
        grid=(B, T // _TT),
        in_specs=[
            pl.BlockSpec(memory_space=pltpu.SMEM),
            pl.BlockSpec((1, 1, N), lambda b, t: (b, 0, 0)),
            pl.BlockSpec((1, _TT, N), lambda b, t: (b, t, 0)),
            pl.BlockSpec((D, N), lambda b, t: (0, 0)),
            pl.BlockSpec((D, 1), lambda b, t: (0, 0)),
            pl.BlockSpec((D, N), lambda b, t: (0, 0)),
            pl.BlockSpec((D, region_emb.shape[0]), lambda b, t: (0, 0)),
            pl.BlockSpec((D, eid_emb.shape[0]), lambda b, t: (0, 0)),
        ],
        out_specs=pl.BlockSpec((1, _TT, D, N), lambda b, t: (b, t, 0, 0)),
        out_shape=jax.ShapeDtypeStruct((B, T, D, N), jnp.float32),
        scratch_shapes=[pltpu.VMEM((D, N), jnp.float32)],
    )(eids32, regions3, spikes, wfull, bcol, slott, regembt, eidembt)
    return outt.transpose(0, 1, 3, 2)


# speedup vs baseline: 1.3801x; 1.0372x over previous
import jax
import jax.numpy as jnp
from jax.experimental import pallas as pl
from jax.experimental.pallas import tpu as pltpu

_TT = 16
_NBUF = 4


def _fused_kernel(eids_ref, regions_ref, spikes_ref, wfull_ref, bcol_ref,
                  slott_ref, regembt_ref, eidembt_ref, out_ref, base_ref,
                  obuf_ref, sems):
    b_idx = pl.program_id(0)
    t_idx = pl.program_id(1)
    nt = pl.num_programs(1)
    nsteps = pl.num_programs(0) * nt
    i = b_idx * nt + t_idx
    slot = jax.lax.rem(i, _NBUF)
    tt = obuf_ref.shape[1]
    d, n = base_ref.shape

    @pl.when(t_idx == 0)
    def _build_base():
        regions = regions_ref[0, :, :]
        nregions = regembt_ref.shape[1]
        oht = (jax.lax.broadcasted_iota(jnp.int32, (nregions, n), 0)
               == regions).astype(jnp.float32)
        regt = jnp.dot(regembt_ref[...], oht,
                       preferred_element_type=jnp.float32)

        e = eids_ref[b_idx]
        neids = eidembt_ref.shape[1]
        ohe = (jax.lax.broadcasted_iota(jnp.int32, (neids, 8), 0) == e
               ).astype(jnp.float32)
        evt = jnp.dot(eidembt_ref[...], ohe,
                      preferred_element_type=jnp.float32)

        base_ref[...] = (slott_ref[...] + regt
                         + evt[:, 0:1] + bcol_ref[...])

    dst = out_ref.at[b_idx, pl.ds(t_idx * tt, tt)]

    @pl.when(i >= _NBUF)
    def _wait_slot():
        pltpu.make_async_copy(obuf_ref.at[slot], dst, sems.at[slot]).wait()

    sp = spikes_ref[0]
    obuf_ref[slot] = (sp[:, None, :] * wfull_ref[...][None, :, :]
                      + base_ref[...][None, :, :])
    pltpu.make_async_copy(obuf_ref.at[slot], dst, sems.at[slot]).start()

    @pl.when(i == nsteps - 1)
    def _drain():
        for k in range(_NBUF):
            pltpu.make_async_copy(obuf_ref.at[k], dst, sems.at[k]).wait()


@jax.jit
def kernel(spikes, neuron_regions, eids, w_spike, b_spike, neuron_slot,
           region_emb, eid_emb):
    B, T, N = spikes.shape
    D = neuron_slot.shape[1]
    regions3 = neuron_regions.astype(jnp.int32).reshape(B, 1, N)
    eids32 = eids.astype(jnp.int32)
    wfull = jnp.broadcast_to(w_spike, (D, N))
    bcol = b_spike.reshape(D, 1)
    slott = neuron_slot[:N].T
    regembt = region_emb.T
    eidembt = eid_emb.T

    outt = pl.pallas_call(
        _fused_kernel,
        grid=(B, T // _TT),
        in_specs=[
            pl.BlockSpec(memory_space=pltpu.SMEM),
            pl.BlockSpec((1, 1, N), lambda b, t: (b, 0, 0)),
            pl.BlockSpec((1, _TT, N), lambda b, t: (b, t, 0)),
            pl.BlockSpec((D, N), lambda b, t: (0, 0)),
            pl.BlockSpec((D, 1), lambda b, t: (0, 0)),
            pl.BlockSpec((D, N), lambda b, t: (0, 0)),
            pl.BlockSpec((D, region_emb.shape[0]), lambda b, t: (0, 0)),
            pl.BlockSpec((D, eid_emb.shape[0]), lambda b, t: (0, 0)),
        ],
        out_specs=pl.BlockSpec(memory_space=pltpu.MemorySpace.HBM),
        out_shape=jax.ShapeDtypeStruct((B, T, D, N), jnp.float32),
        scratch_shapes=[
            pltpu.VMEM((D, N), jnp.float32),
            pltpu.VMEM((_NBUF, _TT, D, N), jnp.float32),
            pltpu.SemaphoreType.DMA((_NBUF,)),
        ],
    )(eids32, regions3, spikes, wfull, bcol, slott, regembt, eidembt)
    return outt.transpose(0, 1, 3, 2)
